# baseline (speedup 1.0000x reference)
import functools

import jax
import jax.numpy as jnp
from jax import lax
from jax.experimental import pallas as pl
from jax.experimental.pallas import tpu as pltpu
import jax.experimental.pallas.tpu_sc as plsc

N = 10000
E = 320000
G = 64
D_IN = 128
D_H1 = 128
D_H2 = 64
D_OUT = 64
D = 128
DH = 64

N_PAD = 10240
NC, NS = 2, 16
NW = NC * NS
CH = 128
ROWS_PER_SUB = N_PAD // NS

E2 = E + N
N2 = 88
E2_PAD = N2 * NW * CH
N1 = 80
E1_PAD = N1 * NW * CH

BR = 512
GRID = N_PAD // BR

KB = 2

_mesh = plsc.VectorSubcoreMesh(core_axis_name="c", subcore_axis_name="s")
_sc_params = pltpu.CompilerParams(use_tc_tiling_on_sc=False)


@functools.partial(
    pl.kernel,
    out_type=jax.ShapeDtypeStruct((NC, N_PAD, D), jnp.float32),
    mesh=_mesh,
    scratch_types=[
        pltpu.VMEM((CH, 16), jnp.float32),
        pltpu.VMEM((N1, CH), jnp.int32),
        pltpu.VMEM_SHARED((N_PAD, 16), jnp.float32),
    ],
    compiler_params=_sc_params,
)
def _deg_kernel(dst_hbm, out_hbm, buf, didx, acc):
    c = lax.axis_index("c")
    s = lax.axis_index("s")
    wid = s * NC + c
    zv = jnp.zeros((16,), jnp.float32)

    def zrow(r, _):
        buf[r, pl.ds(0, 16)] = zv
        return 0

    lax.fori_loop(0, CH, zrow, 0)
    r0 = s * ROWS_PER_SUB
    for k in range(ROWS_PER_SUB // CH):
        pltpu.sync_copy(buf, acc.at[pl.ds(r0 + k * CH, CH)])
    ov = jnp.ones((16,), jnp.float32)

    def orow(r, _):
        buf[r, pl.ds(0, 16)] = ov
        return 0

    lax.fori_loop(0, CH, orow, 0)
    pltpu.sync_copy(dst_hbm.at[wid], didx)
    plsc.subcore_barrier()

    def body(j, _):
        pltpu.sync_copy(buf, acc.at[didx.at[j]], add=True)
        return 0

    lax.fori_loop(0, N1, body, 0)
    plsc.subcore_barrier()
    pltpu.sync_copy(acc.at[pl.ds(r0, ROWS_PER_SUB)],
                    out_hbm.at[c, pl.ds(r0, ROWS_PER_SUB), pl.ds(0, 16)])


def _make_agg(n_halves):
    @functools.partial(
        pl.kernel,
        out_type=jax.ShapeDtypeStruct((NC, N_PAD, D), jnp.float32),
        mesh=_mesh,
        scratch_types=[
            pltpu.VMEM((2, KB, 2 * CH), jnp.int32),
            pltpu.VMEM((2, KB, 2 * CH), jnp.int32),
            pltpu.VMEM((KB, 2 * CH, DH), jnp.float32),
            pltpu.VMEM_SHARED((N_PAD, DH), jnp.float32),
            pltpu.VMEM_SHARED((N_PAD, DH), jnp.float32),
            pltpu.SemaphoreType.DMA,
            pltpu.SemaphoreType.DMA,
            pltpu.SemaphoreType.DMA,
        ],
        compiler_params=_sc_params,
    )
    def _agg(hp_hbm, src_hbm, dst_hbm, out_hbm, sidx, didx, rows, tab, acc,
             gsem, ssem, isem):
        c = lax.axis_index("c")
        s = lax.axis_index("s")
        wid = s * NC + c
        zv = jnp.zeros((16,), jnp.float32)

        def zrow(r, _):
            for k in range(DH // 16):
                rows[0, r, pl.ds(k * 16, 16)] = zv
            return 0

        r0 = s * ROWS_PER_SUB

        for h in range(n_halves):
            lax.fori_loop(0, CH, zrow, 0)
            pltpu.sync_copy(
                hp_hbm.at[pl.ds(r0, ROWS_PER_SUB), pl.ds(h * DH, DH)],
                tab.at[pl.ds(r0, ROWS_PER_SUB)])
            for k in range(ROWS_PER_SUB // CH):
                pltpu.sync_copy(rows.at[0, pl.ds(0, CH)],
                                acc.at[pl.ds(r0 + k * CH, CH)])
            for k in range(KB):
                for q in (0, 1):
                    pltpu.sync_copy(src_hbm.at[wid, 2 * k + q],
                                    sidx.at[0, k, pl.ds(q * CH, CH)])
                    pltpu.sync_copy(dst_hbm.at[wid, 2 * k + q],
                                    didx.at[0, k, pl.ds(q * CH, CH)])
            plsc.subcore_barrier()

            def body(gp, _):
                for p in (0, 1):
                    g = gp * 2 + p
                    nbase = jnp.minimum((g + 1) * KB * 2, N2 - KB * 2)
                    ip = []
                    for k in range(KB):
                        for q in (0, 1):
                            ip.append(pltpu.async_copy(
                                src_hbm.at[wid, pl.ds(nbase + 2 * k + q, 1)],
                                sidx.at[1 - p, pl.ds(k, 1), pl.ds(q * CH, CH)],
                                isem))
                            ip.append(pltpu.async_copy(
                                dst_hbm.at[wid, pl.ds(nbase + 2 * k + q, 1)],
                                didx.at[1 - p, pl.ds(k, 1), pl.ds(q * CH, CH)],
                                isem))
                    gd = []
                    for k in range(KB):
                        gd.append(pltpu.async_copy(tab.at[sidx.at[p, k]],
                                                   rows.at[k], gsem))
                    sd = []
                    for k in range(KB):
                        gd[k].wait()
                        sd.append(pltpu.async_copy(rows.at[k],
                                                   acc.at[didx.at[p, k]],
                                                   ssem, add=True))
                    for k in range(KB):
                        sd[k].wait()
                    for d_ in ip:
                        d_.wait()
                return 0

            lax.fori_loop(0, N2 // (4 * KB), body, 0)
            plsc.subcore_barrier()
            pltpu.sync_copy(
                acc.at[pl.ds(r0, ROWS_PER_SUB)],
                out_hbm.at[c, pl.ds(r0, ROWS_PER_SUB), pl.ds(h * DH, DH)])
            if h + 1 < n_halves:
                plsc.subcore_barrier()

    return _agg


_agg_full = _make_agg(2)
_agg_half = _make_agg(1)


def _mm1_body(d0_ref, d1_ref, x_ref, w_ref, hp_ref, dinv_ref):
    deg = d0_ref[...][:, 0:1] + d1_ref[...][:, 0:1] + 1.0
    dinv = lax.rsqrt(deg)
    dinv_ref[...] = dinv
    hp_ref[...] = jnp.dot(x_ref[...], w_ref[...],
                          preferred_element_type=jnp.float32) * dinv


_mm1 = pl.pallas_call(
    _mm1_body,
    grid=(GRID,),
    in_specs=[
        pl.BlockSpec((BR, D), lambda i: (i, 0)),
        pl.BlockSpec((BR, D), lambda i: (i, 0)),
        pl.BlockSpec((BR, D_IN), lambda i: (i, 0)),
        pl.BlockSpec((D_IN, D_H1), lambda i: (0, 0)),
    ],
    out_specs=[
        pl.BlockSpec((BR, D_H1), lambda i: (i, 0)),
        pl.BlockSpec((BR, 1), lambda i: (i, 0)),
    ],
    out_shape=[
        jax.ShapeDtypeStruct((N_PAD, D_H1), jnp.float32),
        jax.ShapeDtypeStruct((N_PAD, 1), jnp.float32),
    ],
)


def _make_mid(din):
    def body(p0_ref, p1_ref, dinv_ref, b_ref, w_ref, out_ref):
        h = jnp.maximum(
            (p0_ref[...] + p1_ref[...]) * dinv_ref[...] + b_ref[...], 0.0)
        out_ref[...] = jnp.dot(h, w_ref[...],
                               preferred_element_type=jnp.float32) * dinv_ref[...]

    return pl.pallas_call(
        body,
        grid=(GRID,),
        in_specs=[
            pl.BlockSpec((BR, din), lambda i: (i, 0)),
            pl.BlockSpec((BR, din), lambda i: (i, 0)),
            pl.BlockSpec((BR, 1), lambda i: (i, 0)),
            pl.BlockSpec((1, din), lambda i: (0, 0)),
            pl.BlockSpec((din, D), lambda i: (0, 0)),
        ],
        out_specs=pl.BlockSpec((BR, D), lambda i: (i, 0)),
        out_shape=jax.ShapeDtypeStruct((N_PAD, D), jnp.float32),
    )


_mid2 = _make_mid(D)
_mid3 = _make_mid(DH)


def _pool_body(p0_ref, p1_ref, dinv_ref, b_ref, batch_ref, out_ref,
               acc_s, acc_c):
    i = pl.program_id(0)
    h = jnp.maximum((p0_ref[...] + p1_ref[...]) * dinv_ref[...] + b_ref[...],
                    0.0)
    gid = lax.broadcasted_iota(jnp.int32, (BR, G), 1)
    p = jnp.where(batch_ref[...] == gid, 1.0, 0.0)
    ps = lax.dot_general(p, h, (((0,), (0,)), ((), ())),
                         preferred_element_type=jnp.float32)
    pc = lax.dot_general(p, jnp.ones((BR, 1), jnp.float32),
                         (((0,), (0,)), ((), ())),
                         preferred_element_type=jnp.float32)

    @pl.when(i == 0)
    def _():
        acc_s[...] = jnp.zeros_like(acc_s)
        acc_c[...] = jnp.zeros_like(acc_c)

    acc_s[...] += ps
    acc_c[...] += pc

    @pl.when(i == GRID - 1)
    def _():
        out_ref[...] = acc_s[...] / jnp.maximum(acc_c[...], 1.0)


_pool = pl.pallas_call(
    _pool_body,
    grid=(GRID,),
    in_specs=[
        pl.BlockSpec((BR, D_OUT), lambda i: (i, 0)),
        pl.BlockSpec((BR, D_OUT), lambda i: (i, 0)),
        pl.BlockSpec((BR, 1), lambda i: (i, 0)),
        pl.BlockSpec((1, D_OUT), lambda i: (0, 0)),
        pl.BlockSpec((BR, 1), lambda i: (i, 0)),
    ],
    out_specs=pl.BlockSpec((G, D_OUT), lambda i: (0, 0)),
    out_shape=jax.ShapeDtypeStruct((G, D_OUT), jnp.float32),
    scratch_shapes=[
        pltpu.VMEM((G, D_OUT), jnp.float32),
        pltpu.VMEM((G, 1), jnp.float32),
    ],
)


def _pad_w(w):
    return jnp.pad(w, ((0, 0), (0, D - w.shape[1])))


def kernel(x, edge_index, batch, W1, b1, W2, b2, W3, b3):
    src = edge_index[0].astype(jnp.int32)
    dst = edge_index[1].astype(jnp.int32)
    loop = jnp.arange(N, dtype=jnp.int32)
    pad2 = jnp.full((E2_PAD - E2,), N_PAD - 1, jnp.int32)
    src2 = jnp.concatenate([src, loop, pad2]).reshape(NW, N2, CH)
    dst2 = jnp.concatenate([dst, loop, pad2]).reshape(NW, N2, CH)
    pad1 = jnp.full((E1_PAD - E,), N_PAD - 1, jnp.int32)
    dst1 = jnp.concatenate([dst, pad1]).reshape(NW, N1, CH)

    xp = jnp.pad(x, ((0, N_PAD - N), (0, 0)))
    batchp = jnp.pad(batch.astype(jnp.int32), (0, N_PAD - N),
                     constant_values=G).reshape(N_PAD, 1)

    degp = _deg_kernel(dst1)
    hp1, dinv = _mm1(degp[0], degp[1], xp, W1)
    o1 = _agg_full(hp1, src2, dst2)
    hp2 = _mid2(o1[0], o1[1], dinv, jnp.pad(b1, (0, D - D_H1)).reshape(1, D),
                _pad_w(W2))
    o2 = _agg_half(hp2, src2, dst2)
    hp3 = _mid3(o2[0][:, :DH], o2[1][:, :DH], dinv, b2.reshape(1, DH),
                _pad_w(W3))
    o3 = _agg_half(hp3, src2, dst2)
    return _pool(o3[0][:, :D_OUT], o3[1][:, :D_OUT], dinv,
                 b3.reshape(1, D_OUT), batchp)


# speedup vs baseline: 1.1040x; 1.1040x over previous
import functools

import jax
import jax.numpy as jnp
from jax import lax
from jax.experimental import pallas as pl
from jax.experimental.pallas import tpu as pltpu
import jax.experimental.pallas.tpu_sc as plsc

N = 10000
E = 320000
G = 64
D_IN = 128
D_H1 = 128
D_H2 = 64
D_OUT = 64
D = 128
DH = 64

N_PAD = 10240
NC, NS = 2, 16
NW = NC * NS
CH = 128
ROWS_PER_SUB = N_PAD // NS

E2 = E + N
N2 = 88
E2_PAD = N2 * NW * CH
N1 = 80
E1_PAD = N1 * NW * CH

BR = 512
GRID = N_PAD // BR

KB = 4

_mesh = plsc.VectorSubcoreMesh(core_axis_name="c", subcore_axis_name="s")
_sc_params = pltpu.CompilerParams(use_tc_tiling_on_sc=False)


@functools.partial(
    pl.kernel,
    out_type=jax.ShapeDtypeStruct((NC, N_PAD, D), jnp.float32),
    mesh=_mesh,
    scratch_types=[
        pltpu.VMEM((CH, 16), jnp.float32),
        pltpu.VMEM((N1, CH), jnp.int32),
        pltpu.VMEM_SHARED((N_PAD, 16), jnp.float32),
    ],
    compiler_params=_sc_params,
)
def _deg_kernel(dst_hbm, out_hbm, buf, didx, acc):
    c = lax.axis_index("c")
    s = lax.axis_index("s")
    wid = s * NC + c
    zv = jnp.zeros((16,), jnp.float32)

    def zrow(r, _):
        buf[r, pl.ds(0, 16)] = zv
        return 0

    lax.fori_loop(0, CH, zrow, 0)
    r0 = s * ROWS_PER_SUB
    for k in range(ROWS_PER_SUB // CH):
        pltpu.sync_copy(buf, acc.at[pl.ds(r0 + k * CH, CH)])
    ov = jnp.ones((16,), jnp.float32)

    def orow(r, _):
        buf[r, pl.ds(0, 16)] = ov
        return 0

    lax.fori_loop(0, CH, orow, 0)
    pltpu.sync_copy(dst_hbm.at[wid], didx)
    plsc.subcore_barrier()

    def body(j, _):
        pltpu.sync_copy(buf, acc.at[didx.at[j]], add=True)
        return 0

    lax.fori_loop(0, N1, body, 0)
    plsc.subcore_barrier()
    pltpu.sync_copy(acc.at[pl.ds(r0, ROWS_PER_SUB)],
                    out_hbm.at[c, pl.ds(r0, ROWS_PER_SUB), pl.ds(0, 16)])


def _make_agg(n_halves):
    @functools.partial(
        pl.kernel,
        out_type=jax.ShapeDtypeStruct((NC, N_PAD, D), jnp.float32),
        mesh=_mesh,
        scratch_types=[
            pltpu.VMEM((2, KB, CH), jnp.int32),
            pltpu.VMEM((2, KB, CH), jnp.int32),
            pltpu.VMEM((KB, CH, DH), jnp.float32),
            pltpu.VMEM_SHARED((N_PAD, DH), jnp.float32),
            pltpu.VMEM_SHARED((N_PAD, DH), jnp.float32),
            pltpu.SemaphoreType.DMA,
            pltpu.SemaphoreType.DMA,
            pltpu.SemaphoreType.DMA,
        ],
        compiler_params=_sc_params,
    )
    def _agg(hp_hbm, src_hbm, dst_hbm, out_hbm, sidx, didx, rows, tab, acc,
             gsem, ssem, isem):
        c = lax.axis_index("c")
        s = lax.axis_index("s")
        wid = s * NC + c
        zv = jnp.zeros((16,), jnp.float32)

        def zrow(r, _):
            for k in range(DH // 16):
                rows[0, r, pl.ds(k * 16, 16)] = zv
            return 0

        r0 = s * ROWS_PER_SUB

        for h in range(n_halves):
            lax.fori_loop(0, CH, zrow, 0)
            pltpu.sync_copy(
                hp_hbm.at[pl.ds(r0, ROWS_PER_SUB), pl.ds(h * DH, DH)],
                tab.at[pl.ds(r0, ROWS_PER_SUB)])
            for k in range(ROWS_PER_SUB // CH):
                pltpu.sync_copy(rows.at[0], acc.at[pl.ds(r0 + k * CH, CH)])
            pltpu.sync_copy(src_hbm.at[wid, pl.ds(0, KB)], sidx.at[0])
            pltpu.sync_copy(dst_hbm.at[wid, pl.ds(0, KB)], didx.at[0])
            plsc.subcore_barrier()

            def body(gp, _):
                for p in (0, 1):
                    g = gp * 2 + p
                    nbase = jnp.minimum((g + 1) * KB, N2 - KB)
                    ip = [
                        pltpu.async_copy(src_hbm.at[wid, pl.ds(nbase, KB)],
                                         sidx.at[1 - p], isem),
                        pltpu.async_copy(dst_hbm.at[wid, pl.ds(nbase, KB)],
                                         didx.at[1 - p], isem),
                    ]
                    gd = []
                    for k in range(KB):
                        gd.append(pltpu.async_copy(tab.at[sidx.at[p, k]],
                                                   rows.at[k], gsem))
                    sd = []
                    for k in range(KB):
                        gd[k].wait()
                        sd.append(pltpu.async_copy(rows.at[k],
                                                   acc.at[didx.at[p, k]],
                                                   ssem, add=True))
                    for k in range(KB):
                        sd[k].wait()
                    ip[0].wait()
                    ip[1].wait()
                return 0

            lax.fori_loop(0, N2 // (2 * KB), body, 0)
            plsc.subcore_barrier()
            pltpu.sync_copy(
                acc.at[pl.ds(r0, ROWS_PER_SUB)],
                out_hbm.at[c, pl.ds(r0, ROWS_PER_SUB), pl.ds(h * DH, DH)])
            if h + 1 < n_halves:
                plsc.subcore_barrier()

    return _agg


_agg_full = _make_agg(2)
_agg_half = _make_agg(1)


def _mm1a_body(x_ref, w_ref, xw_ref):
    xw_ref[...] = jnp.dot(x_ref[...], w_ref[...],
                          preferred_element_type=jnp.float32)


_mm1a = pl.pallas_call(
    _mm1a_body,
    grid=(GRID,),
    in_specs=[
        pl.BlockSpec((BR, D_IN), lambda i: (i, 0)),
        pl.BlockSpec((D_IN, D_H1), lambda i: (0, 0)),
    ],
    out_specs=pl.BlockSpec((BR, D_H1), lambda i: (i, 0)),
    out_shape=jax.ShapeDtypeStruct((N_PAD, D_H1), jnp.float32),
)


def _mm1b_body(d0_ref, d1_ref, xw_ref, hp_ref, dinv_ref):
    deg = d0_ref[...][:, 0:1] + d1_ref[...][:, 0:1] + 1.0
    dinv = lax.rsqrt(deg)
    dinv_ref[...] = dinv
    hp_ref[...] = xw_ref[...] * dinv


_mm1b = pl.pallas_call(
    _mm1b_body,
    grid=(GRID,),
    in_specs=[
        pl.BlockSpec((BR, D), lambda i: (i, 0)),
        pl.BlockSpec((BR, D), lambda i: (i, 0)),
        pl.BlockSpec((BR, D_H1), lambda i: (i, 0)),
    ],
    out_specs=[
        pl.BlockSpec((BR, D_H1), lambda i: (i, 0)),
        pl.BlockSpec((BR, 1), lambda i: (i, 0)),
    ],
    out_shape=[
        jax.ShapeDtypeStruct((N_PAD, D_H1), jnp.float32),
        jax.ShapeDtypeStruct((N_PAD, 1), jnp.float32),
    ],
)


def _make_mid(din):
    def body(p0_ref, p1_ref, dinv_ref, b_ref, w_ref, out_ref):
        h = jnp.maximum(
            (p0_ref[...] + p1_ref[...]) * dinv_ref[...] + b_ref[...], 0.0)
        out_ref[...] = jnp.dot(h, w_ref[...],
                               preferred_element_type=jnp.float32) * dinv_ref[...]

    return pl.pallas_call(
        body,
        grid=(GRID,),
        in_specs=[
            pl.BlockSpec((BR, din), lambda i: (i, 0)),
            pl.BlockSpec((BR, din), lambda i: (i, 0)),
            pl.BlockSpec((BR, 1), lambda i: (i, 0)),
            pl.BlockSpec((1, din), lambda i: (0, 0)),
            pl.BlockSpec((din, D), lambda i: (0, 0)),
        ],
        out_specs=pl.BlockSpec((BR, D), lambda i: (i, 0)),
        out_shape=jax.ShapeDtypeStruct((N_PAD, D), jnp.float32),
    )


_mid2 = _make_mid(D)
_mid3 = _make_mid(DH)


def _pool_body(p0_ref, p1_ref, dinv_ref, b_ref, batch_ref, out_ref,
               acc_s, acc_c):
    i = pl.program_id(0)
    h = jnp.maximum((p0_ref[...] + p1_ref[...]) * dinv_ref[...] + b_ref[...],
                    0.0)
    gid = lax.broadcasted_iota(jnp.int32, (BR, G), 1)
    p = jnp.where(batch_ref[...] == gid, 1.0, 0.0)
    ps = lax.dot_general(p, h, (((0,), (0,)), ((), ())),
                         preferred_element_type=jnp.float32)
    pc = lax.dot_general(p, jnp.ones((BR, 1), jnp.float32),
                         (((0,), (0,)), ((), ())),
                         preferred_element_type=jnp.float32)

    @pl.when(i == 0)
    def _():
        acc_s[...] = jnp.zeros_like(acc_s)
        acc_c[...] = jnp.zeros_like(acc_c)

    acc_s[...] += ps
    acc_c[...] += pc

    @pl.when(i == GRID - 1)
    def _():
        out_ref[...] = acc_s[...] / jnp.maximum(acc_c[...], 1.0)


_pool = pl.pallas_call(
    _pool_body,
    grid=(GRID,),
    in_specs=[
        pl.BlockSpec((BR, D_OUT), lambda i: (i, 0)),
        pl.BlockSpec((BR, D_OUT), lambda i: (i, 0)),
        pl.BlockSpec((BR, 1), lambda i: (i, 0)),
        pl.BlockSpec((1, D_OUT), lambda i: (0, 0)),
        pl.BlockSpec((BR, 1), lambda i: (i, 0)),
    ],
    out_specs=pl.BlockSpec((G, D_OUT), lambda i: (0, 0)),
    out_shape=jax.ShapeDtypeStruct((G, D_OUT), jnp.float32),
    scratch_shapes=[
        pltpu.VMEM((G, D_OUT), jnp.float32),
        pltpu.VMEM((G, 1), jnp.float32),
    ],
)


def _pad_w(w):
    return jnp.pad(w, ((0, 0), (0, D - w.shape[1])))


def kernel(x, edge_index, batch, W1, b1, W2, b2, W3, b3):
    src = edge_index[0].astype(jnp.int32)
    dst = edge_index[1].astype(jnp.int32)
    loop = jnp.arange(N, dtype=jnp.int32)
    pad2 = N + jnp.arange(E2_PAD - E2, dtype=jnp.int32) % (N_PAD - N)
    src2 = jnp.concatenate([src, loop, pad2]).reshape(NW, N2, CH)
    dst2 = jnp.concatenate([dst, loop, pad2]).reshape(NW, N2, CH)
    pad1 = N + jnp.arange(E1_PAD - E, dtype=jnp.int32) % (N_PAD - N)
    dst1 = jnp.concatenate([dst, pad1]).reshape(NW, N1, CH)

    xp = jnp.pad(x, ((0, N_PAD - N), (0, 0)))
    batchp = jnp.pad(batch.astype(jnp.int32), (0, N_PAD - N),
                     constant_values=G).reshape(N_PAD, 1)

    xw1 = _mm1a(xp, W1)
    degp = _deg_kernel(dst1)
    hp1, dinv = _mm1b(degp[0], degp[1], xw1)
    o1 = _agg_full(hp1, src2, dst2)
    hp2 = _mid2(o1[0], o1[1], dinv, jnp.pad(b1, (0, D - D_H1)).reshape(1, D),
                _pad_w(W2))
    o2 = _agg_half(hp2, src2, dst2)
    hp3 = _mid3(o2[0][:, :DH], o2[1][:, :DH], dinv, b2.reshape(1, DH),
                _pad_w(W3))
    o3 = _agg_half(hp3, src2, dst2)
    return _pool(o3[0][:, :D_OUT], o3[1][:, :D_OUT], dinv,
                 b3.reshape(1, D_OUT), batchp)


# speedup vs baseline: 1.1952x; 1.0827x over previous
import functools

import jax
import jax.numpy as jnp
from jax import lax
from jax.experimental import pallas as pl
from jax.experimental.pallas import tpu as pltpu
import jax.experimental.pallas.tpu_sc as plsc

N = 10000
E = 320000
G = 64
D_IN = 128
D_H1 = 128
D_H2 = 64
D_OUT = 64
D = 128
DH = 64

N_PAD = 10240
NC, NS = 2, 16
NW = NC * NS
CH = 128
ROWS_PER_SUB = N_PAD // NS

N2 = 80
E2_PAD = N2 * NW * CH
N1 = N2
E1_PAD = E2_PAD

BR = 512
GRID = N_PAD // BR

KB = 4

_mesh = plsc.VectorSubcoreMesh(core_axis_name="c", subcore_axis_name="s")
_sc_params = pltpu.CompilerParams(use_tc_tiling_on_sc=False)


@functools.partial(
    pl.kernel,
    out_type=jax.ShapeDtypeStruct((NC, N_PAD, D), jnp.float32),
    mesh=_mesh,
    scratch_types=[
        pltpu.VMEM((CH, 16), jnp.float32),
        pltpu.VMEM((N1, CH), jnp.int32),
        pltpu.VMEM_SHARED((N_PAD, 16), jnp.float32),
    ],
    compiler_params=_sc_params,
)
def _deg_kernel(dst_hbm, out_hbm, buf, didx, acc):
    c = lax.axis_index("c")
    s = lax.axis_index("s")
    wid = s * NC + c
    zv = jnp.zeros((16,), jnp.float32)

    def zrow(r, _):
        buf[r, pl.ds(0, 16)] = zv
        return 0

    lax.fori_loop(0, CH, zrow, 0)
    r0 = s * ROWS_PER_SUB
    for k in range(ROWS_PER_SUB // CH):
        pltpu.sync_copy(buf, acc.at[pl.ds(r0 + k * CH, CH)])
    ov = jnp.ones((16,), jnp.float32)

    def orow(r, _):
        buf[r, pl.ds(0, 16)] = ov
        return 0

    lax.fori_loop(0, CH, orow, 0)
    pltpu.sync_copy(dst_hbm.at[wid], didx)
    plsc.subcore_barrier()

    def body(j, _):
        pltpu.sync_copy(buf, acc.at[didx.at[j]], add=True)
        return 0

    lax.fori_loop(0, N1, body, 0)
    plsc.subcore_barrier()
    pltpu.sync_copy(acc.at[pl.ds(r0, ROWS_PER_SUB)],
                    out_hbm.at[c, pl.ds(r0, ROWS_PER_SUB), pl.ds(0, 16)])


def _make_agg(n_halves):
    @functools.partial(
        pl.kernel,
        out_type=jax.ShapeDtypeStruct((NC, N_PAD, D), jnp.float32),
        mesh=_mesh,
        scratch_types=[
            pltpu.VMEM((2, KB, CH), jnp.int32),
            pltpu.VMEM((2, KB, CH), jnp.int32),
            pltpu.VMEM((KB, CH, DH), jnp.float32),
            pltpu.VMEM_SHARED((N_PAD, DH), jnp.float32),
            pltpu.VMEM_SHARED((N_PAD, DH), jnp.float32),
            pltpu.SemaphoreType.DMA,
            pltpu.SemaphoreType.DMA,
            pltpu.SemaphoreType.DMA,
        ],
        compiler_params=_sc_params,
    )
    def _agg(hp_hbm, src_hbm, dst_hbm, out_hbm, sidx, didx, rows, tab, acc,
             gsem, ssem, isem):
        c = lax.axis_index("c")
        s = lax.axis_index("s")
        wid = s * NC + c
        zv = jnp.zeros((16,), jnp.float32)

        def zrow(r, _):
            for k in range(DH // 16):
                rows[0, r, pl.ds(k * 16, 16)] = zv
            return 0

        r0 = s * ROWS_PER_SUB

        for h in range(n_halves):
            lax.fori_loop(0, CH, zrow, 0)
            pltpu.sync_copy(
                hp_hbm.at[pl.ds(r0, ROWS_PER_SUB), pl.ds(h * DH, DH)],
                tab.at[pl.ds(r0, ROWS_PER_SUB)])
            for k in range(ROWS_PER_SUB // CH):
                pltpu.sync_copy(rows.at[0], acc.at[pl.ds(r0 + k * CH, CH)])
            pltpu.sync_copy(src_hbm.at[wid, pl.ds(0, KB)], sidx.at[0])
            pltpu.sync_copy(dst_hbm.at[wid, pl.ds(0, KB)], didx.at[0])
            plsc.subcore_barrier()

            def body(gp, _):
                for p in (0, 1):
                    g = gp * 2 + p
                    nbase = jnp.minimum((g + 1) * KB, N2 - KB)
                    ip = [
                        pltpu.async_copy(src_hbm.at[wid, pl.ds(nbase, KB)],
                                         sidx.at[1 - p], isem),
                        pltpu.async_copy(dst_hbm.at[wid, pl.ds(nbase, KB)],
                                         didx.at[1 - p], isem),
                    ]
                    gd = []
                    for k in range(KB):
                        gd.append(pltpu.async_copy(tab.at[sidx.at[p, k]],
                                                   rows.at[k], gsem))
                    sd = []
                    for k in range(KB):
                        gd[k].wait()
                        sd.append(pltpu.async_copy(rows.at[k],
                                                   acc.at[didx.at[p, k]],
                                                   ssem, add=True))
                    for k in range(KB):
                        sd[k].wait()
                    ip[0].wait()
                    ip[1].wait()
                return 0

            lax.fori_loop(0, N2 // (2 * KB), body, 0)
            plsc.subcore_barrier()
            pltpu.sync_copy(
                acc.at[pl.ds(r0, ROWS_PER_SUB)],
                out_hbm.at[c, pl.ds(r0, ROWS_PER_SUB), pl.ds(h * DH, DH)])
            if h + 1 < n_halves:
                plsc.subcore_barrier()

    return _agg


_agg_full = _make_agg(2)
_agg_half = _make_agg(1)


def _mm1a_body(x_ref, w_ref, xw_ref):
    xw_ref[...] = jnp.dot(x_ref[...], w_ref[...],
                          preferred_element_type=jnp.float32)


_mm1a = pl.pallas_call(
    _mm1a_body,
    grid=(GRID,),
    in_specs=[
        pl.BlockSpec((BR, D_IN), lambda i: (i, 0)),
        pl.BlockSpec((D_IN, D_H1), lambda i: (0, 0)),
    ],
    out_specs=pl.BlockSpec((BR, D_H1), lambda i: (i, 0)),
    out_shape=jax.ShapeDtypeStruct((N_PAD, D_H1), jnp.float32),
)


def _mm1b_body(d0_ref, d1_ref, xw_ref, hp_ref, dinv_ref):
    deg = d0_ref[...][:, 0:1] + d1_ref[...][:, 0:1] + 1.0
    dinv = lax.rsqrt(deg)
    dinv_ref[...] = dinv
    hp_ref[...] = xw_ref[...] * dinv


_mm1b = pl.pallas_call(
    _mm1b_body,
    grid=(GRID,),
    in_specs=[
        pl.BlockSpec((BR, D), lambda i: (i, 0)),
        pl.BlockSpec((BR, D), lambda i: (i, 0)),
        pl.BlockSpec((BR, D_H1), lambda i: (i, 0)),
    ],
    out_specs=[
        pl.BlockSpec((BR, D_H1), lambda i: (i, 0)),
        pl.BlockSpec((BR, 1), lambda i: (i, 0)),
    ],
    out_shape=[
        jax.ShapeDtypeStruct((N_PAD, D_H1), jnp.float32),
        jax.ShapeDtypeStruct((N_PAD, 1), jnp.float32),
    ],
)


def _make_mid(din):
    def body(p0_ref, p1_ref, hp_ref, dinv_ref, b_ref, w_ref, out_ref):
        h = jnp.maximum(
            (p0_ref[...] + p1_ref[...] + hp_ref[...]) * dinv_ref[...]
            + b_ref[...], 0.0)
        out_ref[...] = jnp.dot(h, w_ref[...],
                               preferred_element_type=jnp.float32) * dinv_ref[...]

    return pl.pallas_call(
        body,
        grid=(GRID,),
        in_specs=[
            pl.BlockSpec((BR, din), lambda i: (i, 0)),
            pl.BlockSpec((BR, din), lambda i: (i, 0)),
            pl.BlockSpec((BR, din), lambda i: (i, 0)),
            pl.BlockSpec((BR, 1), lambda i: (i, 0)),
            pl.BlockSpec((1, din), lambda i: (0, 0)),
            pl.BlockSpec((din, D), lambda i: (0, 0)),
        ],
        out_specs=pl.BlockSpec((BR, D), lambda i: (i, 0)),
        out_shape=jax.ShapeDtypeStruct((N_PAD, D), jnp.float32),
    )


_mid2 = _make_mid(D)
_mid3 = _make_mid(DH)


def _pool_body(p0_ref, p1_ref, hp_ref, dinv_ref, b_ref, batch_ref, out_ref,
               acc_s, acc_c):
    i = pl.program_id(0)
    h = jnp.maximum(
        (p0_ref[...] + p1_ref[...] + hp_ref[...]) * dinv_ref[...]
        + b_ref[...], 0.0)
    gid = lax.broadcasted_iota(jnp.int32, (BR, G), 1)
    p = jnp.where(batch_ref[...] == gid, 1.0, 0.0)
    ps = lax.dot_general(p, h, (((0,), (0,)), ((), ())),
                         preferred_element_type=jnp.float32)
    pc = lax.dot_general(p, jnp.ones((BR, 1), jnp.float32),
                         (((0,), (0,)), ((), ())),
                         preferred_element_type=jnp.float32)

    @pl.when(i == 0)
    def _():
        acc_s[...] = jnp.zeros_like(acc_s)
        acc_c[...] = jnp.zeros_like(acc_c)

    acc_s[...] += ps
    acc_c[...] += pc

    @pl.when(i == GRID - 1)
    def _():
        out_ref[...] = acc_s[...] / jnp.maximum(acc_c[...], 1.0)


_pool = pl.pallas_call(
    _pool_body,
    grid=(GRID,),
    in_specs=[
        pl.BlockSpec((BR, D_OUT), lambda i: (i, 0)),
        pl.BlockSpec((BR, D_OUT), lambda i: (i, 0)),
        pl.BlockSpec((BR, D_OUT), lambda i: (i, 0)),
        pl.BlockSpec((BR, 1), lambda i: (i, 0)),
        pl.BlockSpec((1, D_OUT), lambda i: (0, 0)),
        pl.BlockSpec((BR, 1), lambda i: (i, 0)),
    ],
    out_specs=pl.BlockSpec((G, D_OUT), lambda i: (0, 0)),
    out_shape=jax.ShapeDtypeStruct((G, D_OUT), jnp.float32),
    scratch_shapes=[
        pltpu.VMEM((G, D_OUT), jnp.float32),
        pltpu.VMEM((G, 1), jnp.float32),
    ],
)


def _pad_w(w):
    return jnp.pad(w, ((0, 0), (0, D - w.shape[1])))


def kernel(x, edge_index, batch, W1, b1, W2, b2, W3, b3):
    src = edge_index[0].astype(jnp.int32)
    dst = edge_index[1].astype(jnp.int32)
    pad2 = N + jnp.arange(E2_PAD - E, dtype=jnp.int32) % (N_PAD - N)
    src2 = jnp.concatenate([src, pad2]).reshape(NW, N2, CH)
    dst2 = jnp.concatenate([dst, pad2]).reshape(NW, N2, CH)
    dst1 = dst2

    xp = jnp.pad(x, ((0, N_PAD - N), (0, 0)))
    batchp = jnp.pad(batch.astype(jnp.int32), (0, N_PAD - N),
                     constant_values=G).reshape(N_PAD, 1)

    xw1 = _mm1a(xp, W1)
    degp = _deg_kernel(dst1)
    hp1, dinv = _mm1b(degp[0], degp[1], xw1)
    o1 = _agg_full(hp1, src2, dst2)
    hp2 = _mid2(o1[0], o1[1], hp1, dinv,
                jnp.pad(b1, (0, D - D_H1)).reshape(1, D), _pad_w(W2))
    o2 = _agg_half(hp2, src2, dst2)
    hp3 = _mid3(o2[0][:, :DH], o2[1][:, :DH], hp2[:, :DH], dinv,
                b2.reshape(1, DH), _pad_w(W3))
    o3 = _agg_half(hp3, src2, dst2)
    return _pool(o3[0][:, :D_OUT], o3[1][:, :D_OUT], hp3[:, :D_OUT], dinv,
                 b3.reshape(1, D_OUT), batchp)


# speedup vs baseline: 1.2224x; 1.0227x over previous
import functools

import jax
import jax.numpy as jnp
from jax import lax
from jax.experimental import pallas as pl
from jax.experimental.pallas import tpu as pltpu
import jax.experimental.pallas.tpu_sc as plsc

N = 10000
E = 320000
G = 64
D_IN = 128
D_H1 = 128
D_H2 = 64
D_OUT = 64
D = 128
DH = 64

N_PAD = 10240
NC, NS = 2, 16
NW = NC * NS
CH = 128
ROWS_PER_SUB = N_PAD // NS

N2 = 80
E2_PAD = N2 * NW * CH
N1 = N2
E1_PAD = E2_PAD

BR = 512
GRID = N_PAD // BR

KB = 5

_mesh = plsc.VectorSubcoreMesh(core_axis_name="c", subcore_axis_name="s")
_sc_params = pltpu.CompilerParams(use_tc_tiling_on_sc=False)


@functools.partial(
    pl.kernel,
    out_type=jax.ShapeDtypeStruct((NC, N_PAD, D), jnp.float32),
    mesh=_mesh,
    scratch_types=[
        pltpu.VMEM((CH, 16), jnp.float32),
        pltpu.VMEM((N1, CH), jnp.int32),
        pltpu.VMEM_SHARED((N_PAD, 16), jnp.float32),
    ],
    compiler_params=_sc_params,
)
def _deg_kernel(dst_hbm, out_hbm, buf, didx, acc):
    c = lax.axis_index("c")
    s = lax.axis_index("s")
    wid = s * NC + c
    zv = jnp.zeros((16,), jnp.float32)

    def zrow(r, _):
        buf[r, pl.ds(0, 16)] = zv
        return 0

    lax.fori_loop(0, CH, zrow, 0)
    r0 = s * ROWS_PER_SUB
    for k in range(ROWS_PER_SUB // CH):
        pltpu.sync_copy(buf, acc.at[pl.ds(r0 + k * CH, CH)])
    ov = jnp.ones((16,), jnp.float32)

    def orow(r, _):
        buf[r, pl.ds(0, 16)] = ov
        return 0

    lax.fori_loop(0, CH, orow, 0)
    pltpu.sync_copy(dst_hbm.at[wid], didx)
    plsc.subcore_barrier()

    def body(j, _):
        pltpu.sync_copy(buf, acc.at[didx.at[j]], add=True)
        return 0

    lax.fori_loop(0, N1, body, 0)
    plsc.subcore_barrier()
    pltpu.sync_copy(acc.at[pl.ds(r0, ROWS_PER_SUB)],
                    out_hbm.at[c, pl.ds(r0, ROWS_PER_SUB), pl.ds(0, 16)])


def _make_agg(n_halves):
    @functools.partial(
        pl.kernel,
        out_type=jax.ShapeDtypeStruct((NC, N_PAD, D), jnp.float32),
        mesh=_mesh,
        scratch_types=[
            pltpu.VMEM((2, KB, CH), jnp.int32),
            pltpu.VMEM((2, KB, CH), jnp.int32),
            pltpu.VMEM((KB, CH, DH), jnp.float32),
            pltpu.VMEM_SHARED((N_PAD, DH), jnp.float32),
            pltpu.VMEM_SHARED((N_PAD, DH), jnp.float32),
            pltpu.SemaphoreType.DMA,
            pltpu.SemaphoreType.DMA,
            pltpu.SemaphoreType.DMA,
        ],
        compiler_params=_sc_params,
    )
    def _agg(hp_hbm, src_hbm, dst_hbm, out_hbm, sidx, didx, rows, tab, acc,
             gsem, ssem, isem):
        c = lax.axis_index("c")
        s = lax.axis_index("s")
        wid = s * NC + c
        zv = jnp.zeros((16,), jnp.float32)

        def zrow(r, _):
            for k in range(DH // 16):
                rows[0, r, pl.ds(k * 16, 16)] = zv
            return 0

        r0 = s * ROWS_PER_SUB

        for h in range(n_halves):
            lax.fori_loop(0, CH, zrow, 0)
            pltpu.sync_copy(
                hp_hbm.at[pl.ds(r0, ROWS_PER_SUB), pl.ds(h * DH, DH)],
                tab.at[pl.ds(r0, ROWS_PER_SUB)])
            for k in range(ROWS_PER_SUB // CH):
                pltpu.sync_copy(rows.at[0], acc.at[pl.ds(r0 + k * CH, CH)])
            pltpu.sync_copy(src_hbm.at[wid, pl.ds(0, KB)], sidx.at[0])
            pltpu.sync_copy(dst_hbm.at[wid, pl.ds(0, KB)], didx.at[0])
            plsc.subcore_barrier()

            def body(gp, _):
                for p in (0, 1):
                    g = gp * 2 + p
                    nbase = jnp.minimum((g + 1) * KB, N2 - KB)
                    ip = [
                        pltpu.async_copy(src_hbm.at[wid, pl.ds(nbase, KB)],
                                         sidx.at[1 - p], isem),
                        pltpu.async_copy(dst_hbm.at[wid, pl.ds(nbase, KB)],
                                         didx.at[1 - p], isem),
                    ]
                    gd = []
                    for k in range(KB):
                        gd.append(pltpu.async_copy(tab.at[sidx.at[p, k]],
                                                   rows.at[k], gsem))
                    sd = []
                    for k in range(KB):
                        gd[k].wait()
                        sd.append(pltpu.async_copy(rows.at[k],
                                                   acc.at[didx.at[p, k]],
                                                   ssem, add=True))
                    for k in range(KB):
                        sd[k].wait()
                    ip[0].wait()
                    ip[1].wait()
                return 0

            lax.fori_loop(0, N2 // (2 * KB), body, 0)
            plsc.subcore_barrier()
            pltpu.sync_copy(
                acc.at[pl.ds(r0, ROWS_PER_SUB)],
                out_hbm.at[c, pl.ds(r0, ROWS_PER_SUB), pl.ds(h * DH, DH)])
            if h + 1 < n_halves:
                plsc.subcore_barrier()

    return _agg


_agg_full = _make_agg(2)
_agg_half = _make_agg(1)


def _mm1a_body(x_ref, w_ref, xw_ref):
    xw_ref[...] = jnp.dot(x_ref[...], w_ref[...],
                          preferred_element_type=jnp.float32)


_mm1a = pl.pallas_call(
    _mm1a_body,
    grid=(GRID,),
    in_specs=[
        pl.BlockSpec((BR, D_IN), lambda i: (i, 0)),
        pl.BlockSpec((D_IN, D_H1), lambda i: (0, 0)),
    ],
    out_specs=pl.BlockSpec((BR, D_H1), lambda i: (i, 0)),
    out_shape=jax.ShapeDtypeStruct((N_PAD, D_H1), jnp.float32),
)


def _mm1b_body(d0_ref, d1_ref, xw_ref, hp_ref, dinv_ref):
    deg = d0_ref[...][:, 0:1] + d1_ref[...][:, 0:1] + 1.0
    dinv = lax.rsqrt(deg)
    dinv_ref[...] = dinv
    hp_ref[...] = xw_ref[...] * dinv


_mm1b = pl.pallas_call(
    _mm1b_body,
    grid=(GRID,),
    in_specs=[
        pl.BlockSpec((BR, D), lambda i: (i, 0)),
        pl.BlockSpec((BR, D), lambda i: (i, 0)),
        pl.BlockSpec((BR, D_H1), lambda i: (i, 0)),
    ],
    out_specs=[
        pl.BlockSpec((BR, D_H1), lambda i: (i, 0)),
        pl.BlockSpec((BR, 1), lambda i: (i, 0)),
    ],
    out_shape=[
        jax.ShapeDtypeStruct((N_PAD, D_H1), jnp.float32),
        jax.ShapeDtypeStruct((N_PAD, 1), jnp.float32),
    ],
)


def _make_mid(din):
    def body(p0_ref, p1_ref, hp_ref, dinv_ref, b_ref, w_ref, out_ref):
        h = jnp.maximum(
            (p0_ref[...] + p1_ref[...] + hp_ref[...]) * dinv_ref[...]
            + b_ref[...], 0.0)
        out_ref[...] = jnp.dot(h, w_ref[...],
                               preferred_element_type=jnp.float32) * dinv_ref[...]

    return pl.pallas_call(
        body,
        grid=(GRID,),
        in_specs=[
            pl.BlockSpec((BR, din), lambda i: (i, 0)),
            pl.BlockSpec((BR, din), lambda i: (i, 0)),
            pl.BlockSpec((BR, din), lambda i: (i, 0)),
            pl.BlockSpec((BR, 1), lambda i: (i, 0)),
            pl.BlockSpec((1, din), lambda i: (0, 0)),
            pl.BlockSpec((din, D), lambda i: (0, 0)),
        ],
        out_specs=pl.BlockSpec((BR, D), lambda i: (i, 0)),
        out_shape=jax.ShapeDtypeStruct((N_PAD, D), jnp.float32),
    )


_mid2 = _make_mid(D)
_mid3 = _make_mid(DH)


def _pool_body(p0_ref, p1_ref, hp_ref, dinv_ref, b_ref, batch_ref, out_ref,
               acc_s, acc_c):
    i = pl.program_id(0)
    h = jnp.maximum(
        (p0_ref[...] + p1_ref[...] + hp_ref[...]) * dinv_ref[...]
        + b_ref[...], 0.0)
    gid = lax.broadcasted_iota(jnp.int32, (BR, G), 1)
    p = jnp.where(batch_ref[...] == gid, 1.0, 0.0)
    ps = lax.dot_general(p, h, (((0,), (0,)), ((), ())),
                         preferred_element_type=jnp.float32)
    pc = lax.dot_general(p, jnp.ones((BR, 1), jnp.float32),
                         (((0,), (0,)), ((), ())),
                         preferred_element_type=jnp.float32)

    @pl.when(i == 0)
    def _():
        acc_s[...] = jnp.zeros_like(acc_s)
        acc_c[...] = jnp.zeros_like(acc_c)

    acc_s[...] += ps
    acc_c[...] += pc

    @pl.when(i == GRID - 1)
    def _():
        out_ref[...] = acc_s[...] / jnp.maximum(acc_c[...], 1.0)


_pool = pl.pallas_call(
    _pool_body,
    grid=(GRID,),
    in_specs=[
        pl.BlockSpec((BR, D_OUT), lambda i: (i, 0)),
        pl.BlockSpec((BR, D_OUT), lambda i: (i, 0)),
        pl.BlockSpec((BR, D_OUT), lambda i: (i, 0)),
        pl.BlockSpec((BR, 1), lambda i: (i, 0)),
        pl.BlockSpec((1, D_OUT), lambda i: (0, 0)),
        pl.BlockSpec((BR, 1), lambda i: (i, 0)),
    ],
    out_specs=pl.BlockSpec((G, D_OUT), lambda i: (0, 0)),
    out_shape=jax.ShapeDtypeStruct((G, D_OUT), jnp.float32),
    scratch_shapes=[
        pltpu.VMEM((G, D_OUT), jnp.float32),
        pltpu.VMEM((G, 1), jnp.float32),
    ],
)


def _pad_w(w):
    return jnp.pad(w, ((0, 0), (0, D - w.shape[1])))


def kernel(x, edge_index, batch, W1, b1, W2, b2, W3, b3):
    src = edge_index[0].astype(jnp.int32)
    dst = edge_index[1].astype(jnp.int32)
    pad2 = N + jnp.arange(E2_PAD - E, dtype=jnp.int32) % (N_PAD - N)
    src2 = jnp.concatenate([src, pad2]).reshape(NW, N2, CH)
    dst2 = jnp.concatenate([dst, pad2]).reshape(NW, N2, CH)
    dst1 = dst2

    xp = jnp.pad(x, ((0, N_PAD - N), (0, 0)))
    batchp = jnp.pad(batch.astype(jnp.int32), (0, N_PAD - N),
                     constant_values=G).reshape(N_PAD, 1)

    xw1 = _mm1a(xp, W1)
    degp = _deg_kernel(dst1)
    hp1, dinv = _mm1b(degp[0], degp[1], xw1)
    o1 = _agg_full(hp1, src2, dst2)
    hp2 = _mid2(o1[0], o1[1], hp1, dinv,
                jnp.pad(b1, (0, D - D_H1)).reshape(1, D), _pad_w(W2))
    o2 = _agg_half(hp2, src2, dst2)
    hp3 = _mid3(o2[0][:, :DH], o2[1][:, :DH], hp2[:, :DH], dinv,
                b2.reshape(1, DH), _pad_w(W3))
    o3 = _agg_half(hp3, src2, dst2)
    return _pool(o3[0][:, :D_OUT], o3[1][:, :D_OUT], hp3[:, :D_OUT], dinv,
                 b3.reshape(1, D_OUT), batchp)
